# Initial kernel scaffold; baseline (speedup 1.0000x reference)
import jax
import jax.numpy as jnp
from jax.experimental import pallas as pl


def kernel(user_ids, pos_book_ids, neg_book_ids, user_embed_W, book_embed_W):
    raise NotImplementedError("write your pallas kernel here")


# speedup vs baseline: 2.6101x; 2.6101x over previous
import functools

import jax
import jax.numpy as jnp
from jax import lax
from jax.experimental import pallas as pl
from jax.experimental.pallas import tpu as pltpu
from jax.experimental.pallas import tpu_sc as plsc

B = 16384
D = 64
K = 5
NS_PER_SAMPLE = K + 1

_info = plsc.get_sparse_core_info()
NC = _info.num_cores
NSUB = _info.num_subcores
L = _info.num_lanes
NW = NC * NSUB

S_PER_TILE = B // NW
CHUNK = 128
NCHUNK = S_PER_TILE // CHUNK
ROWS_B_CHUNK = CHUNK * NS_PER_SAMPLE
GPC = CHUNK // L

_mesh = plsc.VectorSubcoreMesh(core_axis_name="c", subcore_axis_name="s")


@functools.partial(
    pl.kernel,
    mesh=_mesh,
    compiler_params=pltpu.CompilerParams(needs_layout_passes=False, use_tc_tiling_on_sc=False),
    out_type=jax.ShapeDtypeStruct((B * NS_PER_SAMPLE,), jnp.float32),
    scratch_types=[
        pltpu.VMEM((S_PER_TILE,), jnp.int32),
        pltpu.VMEM((S_PER_TILE * NS_PER_SAMPLE,), jnp.int32),
        pltpu.VMEM((2, CHUNK, D), jnp.float32),
        pltpu.VMEM((2, ROWS_B_CHUNK, D), jnp.float32),
        pltpu.VMEM((S_PER_TILE * NS_PER_SAMPLE,), jnp.float32),
        pltpu.SemaphoreType.DMA,
        pltpu.SemaphoreType.DMA,
    ],
)
def _sc_scores(uids_hbm, bids_hbm, uW_hbm, bW_hbm, out_hbm,
               uidx_v, bidx_v, urows_v, brows_v, scores_v, sem0, sem1):
    wid = lax.axis_index("s") * NC + lax.axis_index("c")
    sbase = wid * S_PER_TILE

    pltpu.sync_copy(uids_hbm.at[pl.ds(sbase, S_PER_TILE)], uidx_v)
    pltpu.sync_copy(
        bids_hbm.at[pl.ds(sbase * NS_PER_SAMPLE, S_PER_TILE * NS_PER_SAMPLE)],
        bidx_v)

    sems = (sem0, sem1)

    def fire(ch):
        buf = ch % 2
        hs = [pltpu.async_copy(
            uW_hbm.at[uidx_v.at[pl.ds(ch * CHUNK, CHUNK)]],
            urows_v.at[buf], sems[buf])]
        for j in range(ROWS_B_CHUNK // 128):
            hs.append(pltpu.async_copy(
                bW_hbm.at[bidx_v.at[pl.ds(ch * ROWS_B_CHUNK + j * 128, 128)]],
                brows_v.at[buf, pl.ds(j * 128, 128)], sems[buf]))
        return hs

    lane = lax.iota(jnp.int32, L)
    zeros = jnp.zeros((L,), jnp.float32)

    pending = fire(0)
    for ch in range(NCHUNK):
        nxt = fire(ch + 1) if ch + 1 < NCHUNK else None
        for h in pending:
            h.wait()
        pending = nxt
        buf = ch % 2
        u2d = urows_v.at[buf]
        b2d = brows_v.at[buf]
        for g in range(GPC):
            srow = lane + g * L
            brows_j = [srow * NS_PER_SAMPLE + j for j in range(NS_PER_SAMPLE)]

            def body(d, accs, _srow=srow, _brows=brows_j, _u=u2d, _b=b2d):
                col = (jnp.full((L,), d, dtype=jnp.int32) + lane) & (D - 1)
                u = plsc.load_gather(_u, [_srow, col])
                return tuple(
                    accs[j] + u * plsc.load_gather(_b, [_brows[j], col])
                    for j in range(NS_PER_SAMPLE))

            accs = lax.fori_loop(0, D, body, (zeros,) * NS_PER_SAMPLE)
            sflat = (ch * CHUNK + g * L + lane) * NS_PER_SAMPLE
            for j in range(NS_PER_SAMPLE):
                plsc.store_scatter(scores_v, [sflat + j], accs[j])

    pltpu.sync_copy(
        scores_v,
        out_hbm.at[pl.ds(sbase * NS_PER_SAMPLE, S_PER_TILE * NS_PER_SAMPLE)])


_TC_ROWS = (B * NS_PER_SAMPLE) // 1024


def _loss_body(s_ref, o_ref):
    s = s_ref[...]
    r = lax.broadcasted_iota(jnp.int32, (_TC_ROWS, 1024), 0)
    c = lax.broadcasted_iota(jnp.int32, (_TC_ROWS, 1024), 1)
    j = (r * 1024 + c) % NS_PER_SAMPLE
    x = jnp.where(j == 0, s, -s)
    term = jnp.log(1.0 / (1.0 + jnp.exp(-x)) + 1e-10)
    o_ref[...] = jnp.broadcast_to(-jnp.sum(term) / B, (1, 1))


def kernel(user_ids, pos_book_ids, neg_book_ids, user_embed_W, book_embed_W):
    uids = user_ids.astype(jnp.int32)
    bids = jnp.concatenate(
        [pos_book_ids[:, None], neg_book_ids], axis=1
    ).astype(jnp.int32).reshape(-1)
    scores = _sc_scores(uids, bids, user_embed_W, book_embed_W)
    loss = pl.pallas_call(
        _loss_body,
        out_shape=jax.ShapeDtypeStruct((1, 1), jnp.float32),
    )(scores.reshape(_TC_ROWS, 1024))
    return loss[0, 0]


# speedup vs baseline: 2.8195x; 1.0802x over previous
import functools

import jax
import jax.numpy as jnp
from jax import lax
from jax.experimental import pallas as pl
from jax.experimental.pallas import tpu as pltpu
from jax.experimental.pallas import tpu_sc as plsc

B = 16384
D = 64
K = 5
NS_PER_SAMPLE = K + 1

_info = plsc.get_sparse_core_info()
NC = _info.num_cores
NSUB = _info.num_subcores
L = _info.num_lanes
NW = NC * NSUB

S_PER_TILE = B // NW
CHUNK = 64
NCHUNK = S_PER_TILE // CHUNK
ROWS_B_CHUNK = CHUNK * NS_PER_SAMPLE
DP = 128
GPC = CHUNK // L

_mesh = plsc.VectorSubcoreMesh(core_axis_name="c", subcore_axis_name="s")


@functools.partial(
    pl.kernel,
    mesh=_mesh,
    compiler_params=pltpu.CompilerParams(needs_layout_passes=False, use_tc_tiling_on_sc=True),
    out_type=jax.ShapeDtypeStruct((B * NS_PER_SAMPLE,), jnp.float32),
    scratch_types=[
        pltpu.VMEM((S_PER_TILE,), jnp.int32),
        pltpu.VMEM((S_PER_TILE * NS_PER_SAMPLE,), jnp.int32),
        pltpu.VMEM((2, CHUNK, DP), jnp.float32),
        pltpu.VMEM((2, ROWS_B_CHUNK, DP), jnp.float32),
        pltpu.VMEM((S_PER_TILE * NS_PER_SAMPLE,), jnp.float32),
        pltpu.SemaphoreType.DMA,
        pltpu.SemaphoreType.DMA,
    ],
)
def _sc_scores(uids_hbm, bids_hbm, uW_hbm, bW_hbm, out_hbm,
               uidx_v, bidx_v, urows_v, brows_v, scores_v, sem0, sem1):
    wid = lax.axis_index("s") * NC + lax.axis_index("c")
    sbase = wid * S_PER_TILE

    pltpu.sync_copy(uids_hbm.at[pl.ds(sbase, S_PER_TILE)], uidx_v)
    pltpu.sync_copy(
        bids_hbm.at[pl.ds(sbase * NS_PER_SAMPLE, S_PER_TILE * NS_PER_SAMPLE)],
        bidx_v)

    sems = (sem0, sem1)

    def fire(ch):
        buf = ch % 2
        hs = [pltpu.async_copy(
            uW_hbm.at[uidx_v.at[pl.ds(ch * CHUNK, CHUNK)]],
            urows_v.at[buf], sems[buf])]
        for j in range(ROWS_B_CHUNK // 128):
            hs.append(pltpu.async_copy(
                bW_hbm.at[bidx_v.at[pl.ds(ch * ROWS_B_CHUNK + j * 128, 128)]],
                brows_v.at[buf, pl.ds(j * 128, 128)], sems[buf]))

        return hs

    lane = lax.iota(jnp.int32, L)
    zeros = jnp.zeros((L,), jnp.float32)

    pending = fire(0)
    for ch in range(NCHUNK):
        nxt = fire(ch + 1) if ch + 1 < NCHUNK else None
        for h in pending:
            h.wait()
        pending = nxt
        buf = ch % 2
        u2d = urows_v.at[buf]
        b2d = brows_v.at[buf]
        for g in range(GPC):
            srow = lane + g * L
            brows_j = [srow * NS_PER_SAMPLE + j for j in range(NS_PER_SAMPLE)]

            def body(d, accs, _srow=srow, _brows=brows_j, _u=u2d, _b=b2d):
                col = (jnp.full((L,), d, dtype=jnp.int32) + lane) & (D - 1)
                u = plsc.load_gather(_u, [_srow, col])
                return tuple(
                    accs[j] + u * plsc.load_gather(_b, [_brows[j], col])
                    for j in range(NS_PER_SAMPLE))

            accs = lax.fori_loop(0, D, body, (zeros,) * NS_PER_SAMPLE)
            sflat = (ch * CHUNK + g * L + lane) * NS_PER_SAMPLE
            for j in range(NS_PER_SAMPLE):
                plsc.store_scatter(scores_v, [sflat + j], accs[j])

    pltpu.sync_copy(
        scores_v,
        out_hbm.at[pl.ds(sbase * NS_PER_SAMPLE, S_PER_TILE * NS_PER_SAMPLE)])


_TC_ROWS = (B * NS_PER_SAMPLE) // 1024


def _loss_body(s_ref, o_ref):
    s = s_ref[...]
    r = lax.broadcasted_iota(jnp.int32, (_TC_ROWS, 1024), 0)
    c = lax.broadcasted_iota(jnp.int32, (_TC_ROWS, 1024), 1)
    j = (r * 1024 + c) % NS_PER_SAMPLE
    x = jnp.where(j == 0, s, -s)
    term = jnp.log(1.0 / (1.0 + jnp.exp(-x)) + 1e-10)
    o_ref[...] = jnp.broadcast_to(-jnp.sum(term) / B, (1, 1))


def kernel(user_ids, pos_book_ids, neg_book_ids, user_embed_W, book_embed_W):
    uids = user_ids.astype(jnp.int32)
    bids = jnp.concatenate(
        [pos_book_ids[:, None], neg_book_ids], axis=1
    ).astype(jnp.int32).reshape(-1)
    uW128 = jnp.pad(user_embed_W, ((0, 0), (0, DP - D)))
    bW128 = jnp.pad(book_embed_W, ((0, 0), (0, DP - D)))
    scores = _sc_scores(uids, bids, uW128, bW128)
    loss = pl.pallas_call(
        _loss_body,
        out_shape=jax.ShapeDtypeStruct((1, 1), jnp.float32),
    )(scores.reshape(_TC_ROWS, 1024))
    return loss[0, 0]


# speedup vs baseline: 2.9537x; 1.0476x over previous
import functools

import jax
import jax.numpy as jnp
from jax import lax
from jax.experimental import pallas as pl
from jax.experimental.pallas import tpu as pltpu
from jax.experimental.pallas import tpu_sc as plsc

B = 16384
D = 64
K = 5
NS_PER_SAMPLE = K + 1

_info = plsc.get_sparse_core_info()
NC = _info.num_cores
NSUB = _info.num_subcores
L = _info.num_lanes
NW = NC * NSUB

S_PER_TILE = B // NW
CHUNK = 64
NCHUNK = S_PER_TILE // CHUNK
ROWS_B_CHUNK = CHUNK * NS_PER_SAMPLE
DP = 128
GPC = CHUNK // L

_mesh = plsc.VectorSubcoreMesh(core_axis_name="c", subcore_axis_name="s")


@functools.partial(
    pl.kernel,
    mesh=_mesh,
    compiler_params=pltpu.CompilerParams(needs_layout_passes=False, use_tc_tiling_on_sc=True),
    out_type=jax.ShapeDtypeStruct((B * NS_PER_SAMPLE,), jnp.float32),
    scratch_types=[
        pltpu.VMEM((S_PER_TILE,), jnp.int32),
        pltpu.VMEM((S_PER_TILE * NS_PER_SAMPLE,), jnp.int32),
        pltpu.VMEM((2, CHUNK, DP), jnp.float32),
        pltpu.VMEM((2, ROWS_B_CHUNK, DP), jnp.float32),
        pltpu.VMEM((S_PER_TILE * NS_PER_SAMPLE,), jnp.float32),
        pltpu.SemaphoreType.DMA,
        pltpu.SemaphoreType.DMA,
    ],
)
def _sc_scores(uids_hbm, bids_hbm, uW_hbm, bW_hbm, out_hbm,
               uidx_v, bidx_v, urows_v, brows_v, scores_v, sem0, sem1):
    wid = lax.axis_index("s") * NC + lax.axis_index("c")
    sbase = wid * S_PER_TILE

    pltpu.sync_copy(uids_hbm.at[pl.ds(sbase, S_PER_TILE)], uidx_v)
    pltpu.sync_copy(
        bids_hbm.at[pl.ds(sbase * NS_PER_SAMPLE, S_PER_TILE * NS_PER_SAMPLE)],
        bidx_v)

    sems = (sem0, sem1)

    def fire(ch):
        buf = ch % 2
        hs = [pltpu.async_copy(
            uW_hbm.at[uidx_v.at[pl.ds(ch * CHUNK, CHUNK)]],
            urows_v.at[buf], sems[buf])]
        for j in range(ROWS_B_CHUNK // 128):
            hs.append(pltpu.async_copy(
                bW_hbm.at[bidx_v.at[pl.ds(ch * ROWS_B_CHUNK + j * 128, 128)]],
                brows_v.at[buf, pl.ds(j * 128, 128)], sems[buf]))

        return hs

    lane = lax.iota(jnp.int32, L)
    zeros = jnp.zeros((L,), jnp.float32)

    pending = fire(0)
    for ch in range(NCHUNK):
        nxt = fire(ch + 1) if ch + 1 < NCHUNK else None
        for h in pending:
            h.wait()
        pending = nxt
        buf = ch % 2
        u2d = urows_v.at[buf]
        b2d = brows_v.at[buf]
        for g in range(GPC):
            srow = lane + g * L
            brows_j = [srow * NS_PER_SAMPLE + j for j in range(NS_PER_SAMPLE)]

            def body(d, accs, _srow=srow, _brows=brows_j, _u=u2d, _b=b2d):
                col = (jnp.full((L,), d, dtype=jnp.int32) + lane) & (D - 1)
                u = plsc.load_gather(_u, [_srow, col])
                return tuple(
                    accs[j] + u * plsc.load_gather(_b, [_brows[j], col])
                    for j in range(NS_PER_SAMPLE))

            accs = lax.fori_loop(0, D, body, (zeros,) * NS_PER_SAMPLE)
            sflat = (ch * CHUNK + g * L + lane) * NS_PER_SAMPLE
            for j in range(NS_PER_SAMPLE):
                plsc.store_scatter(scores_v, [sflat + j], accs[j])

    pltpu.sync_copy(
        scores_v,
        out_hbm.at[pl.ds(sbase * NS_PER_SAMPLE, S_PER_TILE * NS_PER_SAMPLE)])


_VB = 2048


def _xpose_body(t_ref, o_ref):
    x = t_ref[...]
    eye = (lax.broadcasted_iota(jnp.int32, (D, D), 0)
           == lax.broadcasted_iota(jnp.int32, (D, D), 1)).astype(jnp.float32)
    xt = lax.dot_general(x, eye, (((0,), (0,)), ((), ())),
                         preferred_element_type=jnp.float32)
    o_ref[:, :D] = xt
    o_ref[:, D:] = jnp.zeros((_VB, DP - D), jnp.float32)


def _to_rowmajor128(table):
    v = table.shape[0]
    return pl.pallas_call(
        _xpose_body,
        grid=((v + _VB - 1) // _VB,),
        in_specs=[pl.BlockSpec((D, _VB), lambda i: (0, i))],
        out_specs=pl.BlockSpec((_VB, DP), lambda i: (i, 0)),
        out_shape=jax.ShapeDtypeStruct((v, DP), jnp.float32),
    )(table.T)


_TC_ROWS = (B * NS_PER_SAMPLE) // 1024


def _loss_body(s_ref, o_ref):
    s = s_ref[...]
    r = lax.broadcasted_iota(jnp.int32, (_TC_ROWS, 1024), 0)
    c = lax.broadcasted_iota(jnp.int32, (_TC_ROWS, 1024), 1)
    j = (r * 1024 + c) % NS_PER_SAMPLE
    x = jnp.where(j == 0, s, -s)
    term = jnp.log(1.0 / (1.0 + jnp.exp(-x)) + 1e-10)
    o_ref[...] = jnp.broadcast_to(-jnp.sum(term) / B, (1, 1))


def kernel(user_ids, pos_book_ids, neg_book_ids, user_embed_W, book_embed_W):
    uids = user_ids.astype(jnp.int32)
    bids = jnp.concatenate(
        [pos_book_ids[:, None], neg_book_ids], axis=1
    ).astype(jnp.int32).reshape(-1)
    uW128 = _to_rowmajor128(user_embed_W)
    bW128 = _to_rowmajor128(book_embed_W)
    scores = _sc_scores(uids, bids, uW128, bW128)
    loss = pl.pallas_call(
        _loss_body,
        out_shape=jax.ShapeDtypeStruct((1, 1), jnp.float32),
    )(scores.reshape(_TC_ROWS, 1024))
    return loss[0, 0]
